# baseline (speedup 1.0000x reference)
import functools

import jax
import jax.numpy as jnp
from jax.experimental import pallas as pl
from jax.experimental.pallas import tpu as pltpu

N = 4096
ROWS = 512
NPANEL = N // ROWS


def _phase1_body(x_ref, w1_ref, b1_ref, w2_ref, a_ref, s2_ref, s1_scr):
    i = pl.program_id(0)

    @pl.when(i == 0)
    def _():
        s1_scr[...] = jnp.dot(x_ref[...], w1_ref[...],
                              preferred_element_type=jnp.float32)

    h = jnp.dot(a_ref[...], s1_scr[...], preferred_element_type=jnp.float32)
    h = jnp.maximum(h + b1_ref[...], 0.0)
    s2_ref[...] = jnp.dot(h, w2_ref[...], preferred_element_type=jnp.float32)


def _phase2_body(s2_ref, b2_ref, a_ref, out_ref):
    out_ref[...] = jnp.dot(a_ref[...], s2_ref[...],
                           preferred_element_type=jnp.float32) + b2_ref[...]


@functools.partial(jax.jit, static_argnums=())
def _gcn_branch(adj, x, w1, b1, w2, b2):
    f_in = x.shape[1]
    h1 = w1.shape[1]
    h2 = w2.shape[1]
    b1r = b1.reshape(1, h1)
    b2r = b2.reshape(1, h2)

    s2 = pl.pallas_call(
        _phase1_body,
        grid=(NPANEL,),
        in_specs=[
            pl.BlockSpec((N, f_in), lambda i: (0, 0)),
            pl.BlockSpec((f_in, h1), lambda i: (0, 0)),
            pl.BlockSpec((1, h1), lambda i: (0, 0)),
            pl.BlockSpec((h1, h2), lambda i: (0, 0)),
            pl.BlockSpec((ROWS, N), lambda i: (i, 0)),
        ],
        out_specs=pl.BlockSpec((ROWS, h2), lambda i: (i, 0)),
        out_shape=jax.ShapeDtypeStruct((N, h2), jnp.float32),
        scratch_shapes=[pltpu.VMEM((N, h1), jnp.float32)],
    )(x, w1, b1r, w2, adj)

    out = pl.pallas_call(
        _phase2_body,
        grid=(NPANEL,),
        in_specs=[
            pl.BlockSpec((N, h2), lambda i: (0, 0)),
            pl.BlockSpec((1, h2), lambda i: (0, 0)),
            pl.BlockSpec((ROWS, N), lambda i: (i, 0)),
        ],
        out_specs=pl.BlockSpec((ROWS, h2), lambda i: (i, 0)),
        out_shape=jax.ShapeDtypeStruct((N, h2), jnp.float32),
        compiler_params=pltpu.CompilerParams(
            dimension_semantics=("parallel",)),
    )(s2, b2r, adj)
    return out


def kernel(drug_graph, drug_sim_feat, dis_graph, disease_sim_feat,
           W1_drug, b1_drug, W2_drug, b2_drug,
           W1_dis, b1_dis, W2_dis, b2_dis):
    emb1 = _gcn_branch(drug_graph, drug_sim_feat, W1_drug, b1_drug,
                       W2_drug, b2_drug)
    emb2 = _gcn_branch(dis_graph, disease_sim_feat, W1_dis, b1_dis,
                       W2_dis, b2_dis)
    return (emb1, emb2)
